# Initial kernel scaffold; baseline (speedup 1.0000x reference)
import jax
import jax.numpy as jnp
from jax.experimental import pallas as pl


def kernel(features, tracks, visibility):
    raise NotImplementedError("write your pallas kernel here")


# speedup vs baseline: 3.7892x; 3.7892x over previous
import functools

import jax
import jax.numpy as jnp
from jax import lax
from jax.experimental import pallas as pl
from jax.experimental.pallas import tpu as pltpu
from jax.experimental.pallas import tpu_sc as plsc

PATCH = 14
NC, NS, L = 2, 16, 16
NW = NC * NS
B, T, M = 4, 12, 512
H = W = 27
P = H * W
D = 768
DCH = D // L
NPTS = B * M
PTS_PER_W = NPTS // NW
GROUP = 4
ROWS_PER_GROUP = GROUP * T
NGROUPS = PTS_PER_W // GROUP
NPAIRS = B * (T - 1) * M


def _point_compute(g, p, buf, vis_v, vtc, vtr):
    base = p * T

    def dc_body(dc, accs):
        off = dc * L
        f0 = buf[base, pl.ds(off, L)]
        prev = f0
        nc, nr = [], []
        for t in range(1, T):
            cur = buf[base + t, pl.ds(off, L)]
            nc.append(accs[t - 1] + jnp.abs(cur - prev))
            nr.append(accs[T - 1 + t - 1] + jnp.abs(cur - f0))
            prev = cur
        return tuple(nc) + tuple(nr)

    zeros = tuple(jnp.zeros((L,), jnp.float32) for _ in range(2 * (T - 1)))
    accs = lax.fori_loop(0, DCH, dc_body, zeros)

    pg = g * GROUP + p
    voff = pg * (T * L)
    v = [vis_v[pl.ds(voff + t * L, L)] for t in range(T)]
    tc = vtc[...]
    tr = vtr[...]
    for t in range(1, T):
        tc = tc + accs[t - 1] * (v[t - 1] * v[t])
        tr = tr + accs[T - 1 + t - 1] * (v[0] * v[t])
    vtc[...] = tc
    vtr[...] = tr


def _body(table, tx, ty, tb, vis, out,
          tx_v, ty_v, tb_v, vis_v, idx2, rows0, rows1, vtc, vtr,
          sem0, sem1):
    w = lax.axis_index("s") * NC + lax.axis_index("c")
    fbase = w * (PTS_PER_W * T)

    pltpu.sync_copy(tx.at[pl.ds(fbase, PTS_PER_W * T)], tx_v)
    pltpu.sync_copy(ty.at[pl.ds(fbase, PTS_PER_W * T)], ty_v)
    pltpu.sync_copy(tb.at[pl.ds(fbase, PTS_PER_W * T)], tb_v)
    pltpu.sync_copy(vis.at[pl.ds(fbase * L, PTS_PER_W * T * L)], vis_v)

    inv = jnp.float32(1.0 / PATCH)
    for k in range(PTS_PER_W * T // L):
        xv = tx_v[pl.ds(k * L, L)]
        yv = ty_v[pl.ds(k * L, L)]
        tbv = tb_v[pl.ds(k * L, L)]
        colv = jnp.clip((xv * inv).astype(jnp.int32), 0, W - 1)
        rowv = jnp.clip((yv * inv).astype(jnp.int32), 0, H - 1)
        rid = tbv + rowv * W + colv
        r, c = (k * L) // ROWS_PER_GROUP, (k * L) % ROWS_PER_GROUP
        idx2[r, pl.ds(c, L)] = rid

    vtc[...] = jnp.zeros((L,), jnp.float32)
    vtr[...] = jnp.zeros((L,), jnp.float32)

    pltpu.async_copy(table.at[idx2.at[0]], rows0, sem0)

    def wait(buf, sem):
        pltpu.make_async_copy(table.at[pl.ds(0, ROWS_PER_GROUP)], buf, sem).wait()

    def gi_body(gi, carry):
        g0 = gi * 2
        wait(rows0, sem0)
        pltpu.async_copy(table.at[idx2.at[g0 + 1]], rows1, sem1)
        for p in range(GROUP):
            _point_compute(g0, p, rows0, vis_v, vtc, vtr)
        wait(rows1, sem1)

        @pl.when(gi < NGROUPS // 2 - 1)
        def _():
            pltpu.async_copy(table.at[idx2.at[g0 + 2]], rows0, sem0)

        for p in range(GROUP):
            _point_compute(g0 + 1, p, rows1, vis_v, vtc, vtr)
        return carry

    lax.fori_loop(0, NGROUPS // 2, gi_body, 0)

    scale = jnp.float32(0.01 / NPAIRS)
    vtc[...] = (vtc[...] + vtr[...]) * scale
    pltpu.sync_copy(vtc, out.at[w])


@jax.jit
def _sc_call(table, tx, ty, tb, visf):
    mesh = plsc.VectorSubcoreMesh(core_axis_name="c", subcore_axis_name="s")
    kfn = functools.partial(
        pl.kernel,
        mesh=mesh,
        out_type=jax.ShapeDtypeStruct((NW, L), jnp.float32),
        scratch_types=[
            pltpu.VMEM((PTS_PER_W * T,), jnp.float32),
            pltpu.VMEM((PTS_PER_W * T,), jnp.float32),
            pltpu.VMEM((PTS_PER_W * T,), jnp.int32),
            pltpu.VMEM((PTS_PER_W * T * L,), jnp.float32),
            pltpu.VMEM((NGROUPS, ROWS_PER_GROUP), jnp.int32),
            pltpu.VMEM((ROWS_PER_GROUP, D), jnp.float32),
            pltpu.VMEM((ROWS_PER_GROUP, D), jnp.float32),
            pltpu.VMEM((L,), jnp.float32),
            pltpu.VMEM((L,), jnp.float32),
            pltpu.SemaphoreType.DMA,
            pltpu.SemaphoreType.DMA,
        ],
    )(_body)
    return kfn(table, tx, ty, tb, visf)


def kernel(features, tracks, visibility):
    BT, Pdim, Ddim = features.shape
    table = features.reshape(BT * Pdim, Ddim)
    txy = tracks.transpose(0, 2, 1, 3)
    tx = txy[..., 0].reshape(-1)
    ty = txy[..., 1].reshape(-1)
    parr = jnp.arange(NPTS, dtype=jnp.int32)
    tb = ((parr[:, None] // M) * T + jnp.arange(T, dtype=jnp.int32)[None, :]) * P
    tb = tb.reshape(-1)
    visf = visibility.astype(jnp.float32).transpose(0, 2, 1).reshape(-1)
    visf = jnp.broadcast_to(visf[:, None], (NPTS * T, L)).reshape(-1)
    out = _sc_call(table, tx, ty, tb, visf)
    return jnp.sum(out)


# speedup vs baseline: 12.2648x; 3.2367x over previous
import functools

import jax
import jax.numpy as jnp
from jax import lax
from jax.experimental import pallas as pl
from jax.experimental.pallas import tpu as pltpu
from jax.experimental.pallas import tpu_sc as plsc

PATCH = 14
NC, NS, L = 2, 16, 16
NW = NC * NS
B, T, M = 4, 12, 512
H = W = 27
P = H * W
D = 768
DCH = D // L
NPTS = B * M
PTS_PER_W = NPTS // NW
GROUP = 4
ROWS_PER_GROUP = GROUP * T
NGROUPS = PTS_PER_W // GROUP
GPB = 4
NPAIRS = B * (T - 1) * M
WPB = NW // B


def _point_compute(gg, p, buf, v16, vtc, vtr):
    base = p * T

    def dc_body(dc, accs):
        off = dc * L
        f0 = buf[base, pl.ds(off, L)]
        prev = f0
        nc, nr = [], []
        for t in range(1, T):
            cur = buf[base + t, pl.ds(off, L)]
            nc.append(accs[t - 1] + jnp.abs(cur - prev))
            nr.append(accs[T - 1 + t - 1] + jnp.abs(cur - f0))
            prev = cur
        return tuple(nc) + tuple(nr)

    zeros = tuple(jnp.zeros((L,), jnp.float32) for _ in range(2 * (T - 1)))
    accs = lax.fori_loop(0, DCH, dc_body, zeros)

    lane = gg * GROUP + p
    v = [v16[t][lane] for t in range(T)]
    tc = vtc[...]
    tr = vtr[...]
    for t in range(1, T):
        tc = tc + accs[t - 1] * (v[t - 1] * v[t])
        tr = tr + accs[T - 1 + t - 1] * (v[0] * v[t])
    vtc[...] = tc
    vtr[...] = tr


def _body(table, tx, ty, vis, out,
          tx_v, ty_v, visv, idx1, rows0, rows1, vtc, vtr,
          sem0, sem1):
    w = lax.axis_index("s") * NC + lax.axis_index("c")
    b = w // WPB
    wb = w - b * WPB
    fbase = w * (PTS_PER_W * T)
    bscale = b * T
    m0a = pl.multiple_of((wb // 2) * (2 * PTS_PER_W), 2 * PTS_PER_W)
    coff = (wb - (wb // 2) * 2) * PTS_PER_W
    r_vis = pl.multiple_of(b * 16, 8)

    pltpu.sync_copy(tx.at[pl.ds(fbase, PTS_PER_W * T)], tx_v)
    pltpu.sync_copy(ty.at[pl.ds(fbase, PTS_PER_W * T)], ty_v)
    pltpu.sync_copy(vis.at[pl.ds(r_vis, 16), pl.ds(m0a, 2 * PTS_PER_W)], visv)

    inv = jnp.float32(1.0 / PATCH)
    for k in range(PTS_PER_W * T // L):
        xv = tx_v[pl.ds(k * L, L)]
        yv = ty_v[pl.ds(k * L, L)]
        tv = lax.iota(jnp.int32, L) + jnp.int32((k * L) % T)
        tv = jnp.where(tv >= T, tv - T, tv)
        tv = jnp.where(tv >= T, tv - T, tv)
        colv = jnp.clip((xv * inv).astype(jnp.int32), 0, W - 1)
        rowv = jnp.clip((yv * inv).astype(jnp.int32), 0, H - 1)
        rid = (rowv * W + colv) * (B * T) + (tv + bscale)
        idx1[pl.ds(k * L, L)] = rid

    vtc[...] = jnp.zeros((L,), jnp.float32)
    vtr[...] = jnp.zeros((L,), jnp.float32)

    def start(g, buf, sem):
        off = pl.multiple_of(g * ROWS_PER_GROUP, 8)
        pltpu.async_copy(table.at[idx1.at[pl.ds(off, ROWS_PER_GROUP)]],
                         buf, sem)

    def wait(buf, sem):
        pltpu.make_async_copy(table.at[pl.ds(0, ROWS_PER_GROUP)], buf, sem).wait()

    start(0, rows0, sem0)
    start(1, rows1, sem1)

    def gi_body(gi, carry):
        g0 = gi * GPB
        v16 = [visv[t, pl.ds(coff + gi * L, L)] for t in range(T)]
        for gg in range(GPB):
            buf, sem = (rows0, sem0) if gg % 2 == 0 else (rows1, sem1)
            wait(buf, sem)
            for p in range(GROUP):
                _point_compute(gg, p, buf, v16, vtc, vtr)
            nxt = g0 + gg + 2
            if gg >= GPB - 2:
                @pl.when(gi < NGROUPS // GPB - 1)
                def _():
                    start(nxt, buf, sem)
            else:
                start(nxt, buf, sem)
        return carry

    lax.fori_loop(0, NGROUPS // GPB, gi_body, 0)

    scale = jnp.float32(0.01 / NPAIRS)
    vtc[...] = (vtc[...] + vtr[...]) * scale
    pltpu.sync_copy(vtc, out.at[w])


@jax.jit
def _sc_call(table, tx, ty, vis):
    mesh = plsc.VectorSubcoreMesh(core_axis_name="c", subcore_axis_name="s")
    kfn = functools.partial(
        pl.kernel,
        mesh=mesh,
        out_type=jax.ShapeDtypeStruct((NW, L), jnp.float32),
        scratch_types=[
            pltpu.VMEM((PTS_PER_W * T,), jnp.float32),
            pltpu.VMEM((PTS_PER_W * T,), jnp.float32),
            pltpu.VMEM((16, 2 * PTS_PER_W), jnp.float32),
            pltpu.VMEM((PTS_PER_W * T,), jnp.int32),
            pltpu.VMEM((ROWS_PER_GROUP, D), jnp.float32),
            pltpu.VMEM((ROWS_PER_GROUP, D), jnp.float32),
            pltpu.VMEM((L,), jnp.float32),
            pltpu.VMEM((L,), jnp.float32),
            pltpu.SemaphoreType.DMA,
            pltpu.SemaphoreType.DMA,
        ],
    )(_body)
    return kfn(table, tx, ty, vis)


def kernel(features, tracks, visibility):
    BT, Pdim, Ddim = features.shape
    table = jnp.transpose(features, (1, 0, 2)).reshape(Pdim * BT, Ddim)
    txy = tracks.transpose(0, 2, 1, 3)
    tx = txy[..., 0].reshape(-1)
    ty = txy[..., 1].reshape(-1)
    vis = jnp.pad(visibility.astype(jnp.float32), ((0, 0), (0, 4), (0, 0)))
    vis = vis.reshape(B * 16, M)
    out = _sc_call(table, tx, ty, vis)
    return jnp.sum(out)


# speedup vs baseline: 12.6744x; 1.0334x over previous
import functools

import jax
import jax.numpy as jnp
from jax import lax
from jax.experimental import pallas as pl
from jax.experimental.pallas import tpu as pltpu
from jax.experimental.pallas import tpu_sc as plsc

PATCH = 14
NC, NS, L = 2, 16, 16
NW = NC * NS
B, T, M = 4, 12, 512
H = W = 27
P = H * W
D = 768
DCH = D // L
NPTS = B * M
PTS_PER_W = NPTS // NW
GROUP = 4
ROWS_PER_GROUP = GROUP * T
NGROUPS = PTS_PER_W // GROUP
NPAIRS = B * (T - 1) * M


def _point_compute(g, p, buf, vis_v, vtc, vtr):
    base = p * T

    def dc_body(dc, accs):
        off = dc * L
        f0 = buf[base, pl.ds(off, L)]
        prev = f0
        nc, nr = [], []
        for t in range(1, T):
            cur = buf[base + t, pl.ds(off, L)]
            nc.append(accs[t - 1] + jnp.abs(cur - prev))
            nr.append(accs[T - 1 + t - 1] + jnp.abs(cur - f0))
            prev = cur
        return tuple(nc) + tuple(nr)

    zeros = tuple(jnp.zeros((L,), jnp.float32) for _ in range(2 * (T - 1)))
    accs = lax.fori_loop(0, DCH, dc_body, zeros)

    pg = g * GROUP + p
    voff = pg * T
    v16 = vis_v[pl.ds(voff, L)]
    v = [v16[t] for t in range(T)]
    tc = vtc[...]
    tr = vtr[...]
    for t in range(1, T):
        tc = tc + accs[t - 1] * (v[t - 1] * v[t])
        tr = tr + accs[T - 1 + t - 1] * (v[0] * v[t])
    vtc[...] = tc
    vtr[...] = tr


def _body(table, tx, ty, vis, out,
          tx_v, ty_v, vis_v, idx2, rows0, rows1, vtc, vtr,
          sem0, sem1):
    w = lax.axis_index("s") * NC + lax.axis_index("c")
    fbase = w * (PTS_PER_W * T)
    bscale = (w // (NW // B)) * T

    pltpu.sync_copy(tx.at[pl.ds(fbase, PTS_PER_W * T)], tx_v)
    pltpu.sync_copy(ty.at[pl.ds(fbase, PTS_PER_W * T)], ty_v)
    pltpu.sync_copy(vis.at[pl.ds(fbase, PTS_PER_W * T)],
                    vis_v.at[pl.ds(0, PTS_PER_W * T)])

    inv = jnp.float32(1.0 / PATCH)
    for k in range(PTS_PER_W * T // L):
        xv = tx_v[pl.ds(k * L, L)]
        yv = ty_v[pl.ds(k * L, L)]
        tv = lax.iota(jnp.int32, L) + jnp.int32((k * L) % T)
        tv = jnp.where(tv >= T, tv - T, tv)
        tv = jnp.where(tv >= T, tv - T, tv)
        btv = tv + bscale
        colv = jnp.clip((xv * inv).astype(jnp.int32), 0, W - 1)
        rowv = jnp.clip((yv * inv).astype(jnp.int32), 0, H - 1)
        rid = (rowv * W + colv) * (B * T) + btv
        r, c = (k * L) // ROWS_PER_GROUP, (k * L) % ROWS_PER_GROUP
        idx2[r, pl.ds(c, L)] = rid

    vtc[...] = jnp.zeros((L,), jnp.float32)
    vtr[...] = jnp.zeros((L,), jnp.float32)

    pltpu.async_copy(table.at[idx2.at[0]], rows0, sem0)

    def wait(buf, sem):
        pltpu.make_async_copy(table.at[pl.ds(0, ROWS_PER_GROUP)], buf, sem).wait()

    def gi_body(gi, carry):
        g0 = gi * 2
        wait(rows0, sem0)
        pltpu.async_copy(table.at[idx2.at[g0 + 1]], rows1, sem1)
        for p in range(GROUP):
            _point_compute(g0, p, rows0, vis_v, vtc, vtr)
        wait(rows1, sem1)

        @pl.when(gi < NGROUPS // 2 - 1)
        def _():
            pltpu.async_copy(table.at[idx2.at[g0 + 2]], rows0, sem0)

        for p in range(GROUP):
            _point_compute(g0 + 1, p, rows1, vis_v, vtc, vtr)
        return carry

    lax.fori_loop(0, NGROUPS // 2, gi_body, 0)

    scale = jnp.float32(0.01 / NPAIRS)
    vtc[...] = (vtc[...] + vtr[...]) * scale
    pltpu.sync_copy(vtc, out.at[w])


@jax.jit
def _sc_call(table, tx, ty, visf):
    mesh = plsc.VectorSubcoreMesh(core_axis_name="c", subcore_axis_name="s")
    kfn = functools.partial(
        pl.kernel,
        mesh=mesh,
        out_type=jax.ShapeDtypeStruct((NW, L), jnp.float32),
        scratch_types=[
            pltpu.VMEM((PTS_PER_W * T,), jnp.float32),
            pltpu.VMEM((PTS_PER_W * T,), jnp.float32),
            pltpu.VMEM((PTS_PER_W * T + L,), jnp.float32),
            pltpu.VMEM((NGROUPS, ROWS_PER_GROUP), jnp.int32),
            pltpu.VMEM((ROWS_PER_GROUP, D), jnp.float32),
            pltpu.VMEM((ROWS_PER_GROUP, D), jnp.float32),
            pltpu.VMEM((L,), jnp.float32),
            pltpu.VMEM((L,), jnp.float32),
            pltpu.SemaphoreType.DMA,
            pltpu.SemaphoreType.DMA,
        ],
    )(_body)
    return kfn(table, tx, ty, visf)


def kernel(features, tracks, visibility):
    BT, Pdim, Ddim = features.shape
    table = jnp.transpose(features, (1, 0, 2)).reshape(Pdim * BT, Ddim)
    txy = tracks.transpose(0, 2, 1, 3)
    tx = txy[..., 0].reshape(-1)
    ty = txy[..., 1].reshape(-1)
    visf = visibility.astype(jnp.float32).transpose(0, 2, 1).reshape(-1)
    out = _sc_call(table, tx, ty, visf)
    return jnp.sum(out)


# speedup vs baseline: 12.7821x; 1.0085x over previous
import functools

import jax
import jax.numpy as jnp
from jax import lax
from jax.experimental import pallas as pl
from jax.experimental.pallas import tpu as pltpu
from jax.experimental.pallas import tpu_sc as plsc

PATCH = 14
NC, NS, L = 2, 16, 16
NW = NC * NS
B, T, M = 4, 12, 512
H = W = 27
P = H * W
D = 768
DCH = D // L
NPTS = B * M
PTS_PER_W = NPTS // NW
GROUP = 4
ROWS_PER_GROUP = GROUP * T
NGROUPS = PTS_PER_W // GROUP
NPAIRS = B * (T - 1) * M


def _point_compute(g, p, buf, vis_v, vtc, vtr):
    base = p * T

    def dc_body(dc, accs):
        off = dc * L
        f0 = buf[base, pl.ds(off, L)]
        prev = f0
        nc, nr = [], []
        for t in range(1, T):
            cur = buf[base + t, pl.ds(off, L)]
            nc.append(accs[t - 1] + jnp.abs(cur - prev))
            nr.append(accs[T - 1 + t - 1] + jnp.abs(cur - f0))
            prev = cur
        return tuple(nc) + tuple(nr)

    zeros = tuple(jnp.zeros((L,), jnp.float32) for _ in range(2 * (T - 1)))
    accs = lax.fori_loop(0, DCH, dc_body, zeros)

    pg = g * GROUP + p
    voff = pg * T
    v16 = vis_v[pl.ds(voff, L)]
    v = [v16[t] for t in range(T)]
    tc = vtc[...]
    tr = vtr[...]
    for t in range(1, T):
        tc = tc + accs[t - 1] * (v[t - 1] * v[t])
        tr = tr + accs[T - 1 + t - 1] * (v[0] * v[t])
    vtc[...] = tc
    vtr[...] = tr


def _body(table, tx, ty, vis, out,
          tx_v, ty_v, vis_v, idx2, rows0, rows1, vtc, vtr,
          sem0, sem1):
    w = lax.axis_index("s") * NC + lax.axis_index("c")
    fbase = w * (PTS_PER_W * T)
    bscale = (w // (NW // B)) * T

    pltpu.sync_copy(tx.at[pl.ds(fbase, PTS_PER_W * T)], tx_v)
    pltpu.sync_copy(ty.at[pl.ds(fbase, PTS_PER_W * T)], ty_v)

    inv = jnp.float32(1.0 / PATCH)

    def index_chunk(k):
        xv = tx_v[pl.ds(k * L, L)]
        yv = ty_v[pl.ds(k * L, L)]
        tv = lax.iota(jnp.int32, L) + jnp.int32((k * L) % T)
        tv = jnp.where(tv >= T, tv - T, tv)
        tv = jnp.where(tv >= T, tv - T, tv)
        btv = tv + bscale
        colv = jnp.clip((xv * inv).astype(jnp.int32), 0, W - 1)
        rowv = jnp.clip((yv * inv).astype(jnp.int32), 0, H - 1)
        rid = (rowv * W + colv) * (B * T) + btv
        r, c = (k * L) // ROWS_PER_GROUP, (k * L) % ROWS_PER_GROUP
        idx2[r, pl.ds(c, L)] = rid

    for k in range(ROWS_PER_GROUP // L):
        index_chunk(k)
    pltpu.async_copy(table.at[idx2.at[0]], rows0, sem0)
    for k in range(ROWS_PER_GROUP // L, PTS_PER_W * T // L):
        index_chunk(k)
    pltpu.sync_copy(vis.at[pl.ds(fbase, PTS_PER_W * T)],
                    vis_v.at[pl.ds(0, PTS_PER_W * T)])

    vtc[...] = jnp.zeros((L,), jnp.float32)
    vtr[...] = jnp.zeros((L,), jnp.float32)

    def wait(buf, sem):
        pltpu.make_async_copy(table.at[pl.ds(0, ROWS_PER_GROUP)], buf, sem).wait()

    def gi_body(gi, carry):
        g0 = gi * 2
        wait(rows0, sem0)
        pltpu.async_copy(table.at[idx2.at[g0 + 1]], rows1, sem1)
        for p in range(GROUP):
            _point_compute(g0, p, rows0, vis_v, vtc, vtr)
        wait(rows1, sem1)

        @pl.when(gi < NGROUPS // 2 - 1)
        def _():
            pltpu.async_copy(table.at[idx2.at[g0 + 2]], rows0, sem0)

        for p in range(GROUP):
            _point_compute(g0 + 1, p, rows1, vis_v, vtc, vtr)
        return carry

    lax.fori_loop(0, NGROUPS // 2, gi_body, 0)

    scale = jnp.float32(0.01 / NPAIRS)
    vtc[...] = (vtc[...] + vtr[...]) * scale
    pltpu.sync_copy(vtc, out.at[w])


@jax.jit
def _sc_call(table, tx, ty, visf):
    mesh = plsc.VectorSubcoreMesh(core_axis_name="c", subcore_axis_name="s")
    kfn = functools.partial(
        pl.kernel,
        mesh=mesh,
        out_type=jax.ShapeDtypeStruct((NW, L), jnp.float32),
        scratch_types=[
            pltpu.VMEM((PTS_PER_W * T,), jnp.float32),
            pltpu.VMEM((PTS_PER_W * T,), jnp.float32),
            pltpu.VMEM((PTS_PER_W * T + L,), jnp.float32),
            pltpu.VMEM((NGROUPS, ROWS_PER_GROUP), jnp.int32),
            pltpu.VMEM((ROWS_PER_GROUP, D), jnp.float32),
            pltpu.VMEM((ROWS_PER_GROUP, D), jnp.float32),
            pltpu.VMEM((L,), jnp.float32),
            pltpu.VMEM((L,), jnp.float32),
            pltpu.SemaphoreType.DMA,
            pltpu.SemaphoreType.DMA,
        ],
    )(_body)
    return kfn(table, tx, ty, visf)


def kernel(features, tracks, visibility):
    BT, Pdim, Ddim = features.shape
    table = jnp.transpose(features, (1, 0, 2)).reshape(Pdim * BT, Ddim)
    txy = tracks.transpose(0, 2, 1, 3)
    tx = txy[..., 0].reshape(-1)
    ty = txy[..., 1].reshape(-1)
    visf = visibility.astype(jnp.float32).transpose(0, 2, 1).reshape(-1)
    out = _sc_call(table, tx, ty, visf)
    return jnp.sum(out)
